# baseline (speedup 1.0000x reference)
import jax
import jax.numpy as jnp
from jax import lax
from jax.experimental import pallas as pl
from jax.experimental.pallas import tpu as pltpu
from jax.experimental.pallas import tpu_sc as plsc

B = 16384
K = 20
D = 32
NC = 2
NS = 16
NW = NC * NS
NB = B // NW
NBUF = 8
KP = 24
TCH = 128
NTC = NB // TCH


def _sc_body(tgt_i, ctx_i, negp, inemb, outemb, pos_o, neg_o,
             tgt_iv, ctx_iv, negi_v, tgt_r, ctx_r, pos_v, neg_v,
             sem_tc, *rest):
    nbufs, sems = rest[:NBUF], rest[NBUF:]
    w = lax.axis_index("s") * NC + lax.axis_index("c")

    pltpu.sync_copy(tgt_i.at[pl.ds(w * NB, NB)], tgt_iv)
    pltpu.sync_copy(ctx_i.at[pl.ds(w * NB, NB)], ctx_iv)
    pltpu.sync_copy(negp.at[pl.ds(w * NB, NB), pl.ds(0, KP)], negi_v)

    handles = []
    for j in range(NTC):
        handles.append(pltpu.async_copy(
            inemb.at[tgt_iv.at[pl.ds(j * TCH, TCH)]],
            tgt_r.at[pl.ds(j * TCH, TCH)], sem_tc))
        handles.append(pltpu.async_copy(
            outemb.at[ctx_iv.at[pl.ds(j * TCH, TCH)]],
            ctx_r.at[pl.ds(j * TCH, TCH)], sem_tc))

    for j in range(NBUF):
        pltpu.async_copy(outemb.at[negi_v.at[j]], nbufs[j], sems[j])

    for h in handles:
        h.wait()

    iota = lax.iota(jnp.int32, 16)

    def pos_body(g, carry):
        acc = jnp.zeros((16,), jnp.float32)
        for rr in range(16):
            b = g * 16 + rr
            t0 = tgt_r[b, pl.ds(0, 16)]
            t1 = tgt_r[b, pl.ds(16, 16)]
            c0 = ctx_r[b, pl.ds(0, 16)]
            c1 = ctx_r[b, pl.ds(16, 16)]
            s = jnp.sum(t0 * c0 + t1 * c1)
            acc = jnp.where(iota == rr, s, acc)
        pos_v[pl.ds(g * 16, 16)] = acc
        return carry

    lax.fori_loop(0, NB // 16, pos_body, 0)

    def neg_body(o, carry):
        for j in range(NBUF):
            b = o * NBUF + j
            pltpu.make_async_copy(
                outemb.at[negi_v.at[b]], nbufs[j], sems[j]).wait()
            t0 = tgt_r[b, pl.ds(0, 16)]
            t1 = tgt_r[b, pl.ds(16, 16)]
            acc0 = jnp.zeros((16,), jnp.float32)
            acc1 = jnp.zeros((16,), jnp.float32)
            for rr in range(K):
                n0 = nbufs[j][rr, pl.ds(0, 16)]
                n1 = nbufs[j][rr, pl.ds(16, 16)]
                s = jnp.sum(n0 * t0 + n1 * t1)
                if rr < 16:
                    acc0 = jnp.where(iota == rr, s, acc0)
                else:
                    acc1 = jnp.where(iota == rr - 16, s, acc1)
            neg_v[b, pl.ds(0, 16)] = acc0
            neg_v[b, pl.ds(16, 16)] = acc1
            nxt = (b + NBUF) % NB
            pltpu.async_copy(outemb.at[negi_v.at[nxt]], nbufs[j], sems[j])
        return carry

    lax.fori_loop(0, NB // NBUF, neg_body, 0)

    for j in range(NBUF):
        pltpu.make_async_copy(
            outemb.at[negi_v.at[j]], nbufs[j], sems[j]).wait()

    pltpu.sync_copy(pos_v, pos_o.at[pl.ds(w * NB, NB)])
    pltpu.sync_copy(neg_v, neg_o.at[pl.ds(w * NB, NB), :])


@jax.jit
def _sc_dots(tgt_idx, ctx_idx, negp, in_emb, out_emb):
    mesh = plsc.VectorSubcoreMesh(core_axis_name="c", subcore_axis_name="s")
    f = pl.kernel(
        _sc_body,
        mesh=mesh,
        compiler_params=pltpu.CompilerParams(
            needs_layout_passes=False, use_tc_tiling_on_sc=False),
        out_type=[
            jax.ShapeDtypeStruct((B,), jnp.float32),
            jax.ShapeDtypeStruct((B, D), jnp.float32),
        ],
        scratch_types=(
            [pltpu.VMEM((NB,), jnp.int32),
             pltpu.VMEM((NB,), jnp.int32),
             pltpu.VMEM((NB, KP), jnp.int32),
             pltpu.VMEM((NB, D), jnp.float32),
             pltpu.VMEM((NB, D), jnp.float32),
             pltpu.VMEM((NB,), jnp.float32),
             pltpu.VMEM((NB, D), jnp.float32),
             pltpu.SemaphoreType.DMA]
            + [pltpu.VMEM((KP, D), jnp.float32)] * NBUF
            + [pltpu.SemaphoreType.DMA] * NBUF
        ),
    )
    return f(tgt_idx, ctx_idx, negp, in_emb, out_emb)


def _tc_loss_body(pos_ref, neg_ref, o_ref):
    p = pos_ref[...]
    n = neg_ref[...]

    def ls(v):
        return jnp.minimum(v, 0.0) - jnp.log(1.0 + jnp.exp(-jnp.abs(v)))

    cols = lax.broadcasted_iota(jnp.int32, n.shape, 1)
    valid = (cols % D) < K
    s_pos = jnp.sum(ls(p))
    s_neg = jnp.sum(jnp.where(valid, ls(-n), 0.0))
    o_ref[0, 0] = -(s_pos / B + s_neg / (B * K))


@jax.jit
def _tc_loss(pos2, neg2):
    out = pl.pallas_call(
        _tc_loss_body,
        out_shape=jax.ShapeDtypeStruct((1, 1), jnp.float32),
        out_specs=pl.BlockSpec(memory_space=pltpu.SMEM),
    )(pos2, neg2)
    return out[0, 0]


def kernel(target, context, neg_samples, in_emb, out_emb):
    negi = jnp.pad(neg_samples.astype(jnp.int32), ((0, 0), (0, 128 - K)))
    pos1, negp = _sc_dots(target.astype(jnp.int32), context.astype(jnp.int32),
                          negi, in_emb, out_emb)
    return _tc_loss(pos1.reshape(B // 128, 128),
                    negp.reshape(B * D // 128, 128))


# speedup vs baseline: 1.6492x; 1.6492x over previous
import jax
import jax.numpy as jnp
from jax import lax
from jax.experimental import pallas as pl
from jax.experimental.pallas import tpu as pltpu
from jax.experimental.pallas import tpu_sc as plsc

B = 16384
K = 20
D = 32
NC = 2
NS = 16
NW = NC * NS
NB = B // NW
NBUF = 8
KP = 24
TCH = 128
NTC = NB // TCH


def _sc_body(tgt_i, ctx_i, negp, inemb, outemb, pos_o, neg_o,
             tgt_iv, ctx_iv, negi_v, tgt_r, ctx_r, pos_v, neg_v,
             sem_tc, *rest):
    nbufs, sems = rest[:NBUF], rest[NBUF:]
    w = lax.axis_index("s") * NC + lax.axis_index("c")

    pltpu.sync_copy(tgt_i.at[pl.ds(w * NB, NB)], tgt_iv)
    pltpu.sync_copy(ctx_i.at[pl.ds(w * NB, NB)], ctx_iv)
    pltpu.sync_copy(negp.at[pl.ds(w * NB, NB), :], negi_v)

    handles = []
    for j in range(NTC):
        handles.append(pltpu.async_copy(
            inemb.at[tgt_iv.at[pl.ds(j * TCH, TCH)]],
            tgt_r.at[pl.ds(j * TCH, TCH)], sem_tc))
        handles.append(pltpu.async_copy(
            outemb.at[ctx_iv.at[pl.ds(j * TCH, TCH)]],
            ctx_r.at[pl.ds(j * TCH, TCH)], sem_tc))

    for j in range(NBUF):
        pltpu.async_copy(outemb.at[negi_v.at[j]], nbufs[j], sems[j])

    for h in handles:
        h.wait()

    iota = lax.iota(jnp.int32, 16)

    def pos_body(g, carry):
        acc = jnp.zeros((16,), jnp.float32)
        for rr in range(16):
            b = g * 16 + rr
            t0 = tgt_r[b, pl.ds(0, 16)]
            t1 = tgt_r[b, pl.ds(16, 16)]
            c0 = ctx_r[b, pl.ds(0, 16)]
            c1 = ctx_r[b, pl.ds(16, 16)]
            s = jnp.sum(t0 * c0 + t1 * c1)
            acc = jnp.where(iota == rr, s, acc)
        pos_v[pl.ds(g * 16, 16)] = acc
        return carry

    lax.fori_loop(0, NB // 16, pos_body, 0)

    def neg_body(o, carry):
        for j in range(NBUF):
            b = o * NBUF + j
            pltpu.make_async_copy(
                outemb.at[negi_v.at[b]], nbufs[j], sems[j]).wait()
            t0 = tgt_r[b, pl.ds(0, 16)]
            t1 = tgt_r[b, pl.ds(16, 16)]
            acc0 = jnp.zeros((16,), jnp.float32)
            acc1 = jnp.zeros((16,), jnp.float32)
            for rr in range(K):
                n0 = nbufs[j][rr, pl.ds(0, 16)]
                n1 = nbufs[j][rr, pl.ds(16, 16)]
                s = jnp.sum(n0 * t0 + n1 * t1)
                if rr < 16:
                    acc0 = jnp.where(iota == rr, s, acc0)
                else:
                    acc1 = jnp.where(iota == rr - 16, s, acc1)
            neg_v[b, pl.ds(0, 16)] = acc0
            neg_v[b, pl.ds(16, 16)] = acc1
            nxt = (b + NBUF) % NB
            pltpu.async_copy(outemb.at[negi_v.at[nxt]], nbufs[j], sems[j])
        return carry

    lax.fori_loop(0, NB // NBUF, neg_body, 0)

    for j in range(NBUF):
        pltpu.make_async_copy(
            outemb.at[negi_v.at[j]], nbufs[j], sems[j]).wait()

    pltpu.sync_copy(pos_v, pos_o.at[pl.ds(w * NB, NB)])
    pltpu.sync_copy(neg_v, neg_o.at[pl.ds(w * NB, NB), :])


@jax.jit
def _sc_dots(tgt_idx, ctx_idx, negp, in_emb, out_emb):
    mesh = plsc.VectorSubcoreMesh(core_axis_name="c", subcore_axis_name="s")
    f = pl.kernel(
        _sc_body,
        mesh=mesh,
        compiler_params=pltpu.CompilerParams(
            needs_layout_passes=False, use_tc_tiling_on_sc=False),
        out_type=[
            jax.ShapeDtypeStruct((B,), jnp.float32),
            jax.ShapeDtypeStruct((B, D), jnp.float32),
        ],
        scratch_types=(
            [pltpu.VMEM((NB,), jnp.int32),
             pltpu.VMEM((NB,), jnp.int32),
             pltpu.VMEM((NB, K), jnp.int32),
             pltpu.VMEM((NB, D), jnp.float32),
             pltpu.VMEM((NB, D), jnp.float32),
             pltpu.VMEM((NB,), jnp.float32),
             pltpu.VMEM((NB, D), jnp.float32),
             pltpu.SemaphoreType.DMA]
            + [pltpu.VMEM((K, D), jnp.float32)] * NBUF
            + [pltpu.SemaphoreType.DMA] * NBUF
        ),
    )
    return f(tgt_idx, ctx_idx, negp, in_emb, out_emb)


def _tc_loss_body(pos_ref, neg_ref, o_ref):
    p = pos_ref[...]
    n = neg_ref[...]

    def ls(v):
        return jnp.minimum(v, 0.0) - jnp.log(1.0 + jnp.exp(-jnp.abs(v)))

    cols = lax.broadcasted_iota(jnp.int32, n.shape, 1)
    valid = (cols % D) < K
    s_pos = jnp.sum(ls(p))
    s_neg = jnp.sum(jnp.where(valid, ls(-n), 0.0))
    o_ref[0, 0] = -(s_pos / B + s_neg / (B * K))


@jax.jit
def _tc_loss(pos2, neg2):
    out = pl.pallas_call(
        _tc_loss_body,
        out_shape=jax.ShapeDtypeStruct((1, 1), jnp.float32),
        out_specs=pl.BlockSpec(memory_space=pltpu.SMEM),
    )(pos2, neg2)
    return out[0, 0]


def kernel(target, context, neg_samples, in_emb, out_emb):
    pos1, negp = _sc_dots(target.astype(jnp.int32), context.astype(jnp.int32),
                          neg_samples.astype(jnp.int32), in_emb, out_emb)
    return _tc_loss(pos1.reshape(B // 128, 128),
                    negp.reshape(B * D // 128, 128))


# speedup vs baseline: 2.1459x; 1.3012x over previous
import jax
import jax.numpy as jnp
from jax import lax
from jax.experimental import pallas as pl
from jax.experimental.pallas import tpu as pltpu
from jax.experimental.pallas import tpu_sc as plsc

B = 16384
K = 20
D = 32
NC = 2
NS = 16
NW = NC * NS
NB = B // NW
NBUF = 8
TCH = 128
NTC = NB // TCH
NGRP = NB // 16
NL = NB // 4


def _sc_tgt_body(tgt_i, inemb3, trows_o, idx_v, out_v, *rest):
    tbufs, tsems = rest[:16], rest[16:]
    w = lax.axis_index("s") * NC + lax.axis_index("c")
    pltpu.sync_copy(tgt_i.at[pl.ds(w * NB, NB)], idx_v)
    v0g = idx_v[pl.ds(0, 16)]
    for j in range(16):
        t = v0g[j]
        pltpu.async_copy(inemb3.at[t // 8], tbufs[j], tsems[j])

    def body(g, vg):
        vn = idx_v[pl.ds(((g + 1) % NGRP) * 16, 16)]
        for j in range(16):
            t = vg[j]
            pltpu.make_async_copy(
                inemb3.at[t // 8], tbufs[j], tsems[j]).wait()
            r = t % 8
            v0 = tbufs[j][r, pl.ds(0, 16)]
            v1 = tbufs[j][r, pl.ds(16, 16)]
            line = g * 4 + j // 4
            col = (j % 4) * D
            out_v[line, pl.ds(col, 16)] = v0
            out_v[line, pl.ds(col + 16, 16)] = v1
            t2 = vn[j]
            pltpu.async_copy(inemb3.at[t2 // 8], tbufs[j], tsems[j])
        return vn

    fin = lax.fori_loop(0, NGRP, body, v0g)
    for j in range(16):
        t = fin[j]
        pltpu.make_async_copy(inemb3.at[t // 8], tbufs[j], tsems[j]).wait()
    pltpu.sync_copy(out_v, trows_o.at[pl.ds(w * NL, NL), :])


@jax.jit
def _sc_tgt_rows(target, in_emb3):
    mesh = plsc.VectorSubcoreMesh(core_axis_name="c", subcore_axis_name="s")
    f = pl.kernel(
        _sc_tgt_body,
        mesh=mesh,
        compiler_params=pltpu.CompilerParams(needs_layout_passes=False),
        out_type=jax.ShapeDtypeStruct((B // 4, 128), jnp.float32),
        scratch_types=(
            [pltpu.VMEM((NB,), jnp.int32),
             pltpu.VMEM((NL, 128), jnp.float32)]
            + [pltpu.VMEM((8, D), jnp.float32)] * 16
            + [pltpu.SemaphoreType.DMA] * 16
        ),
    )
    return f(target, in_emb3)


def _sc_body(ctx_i, neg_i, trows, outemb, pos_o, neg_o,
             ctx_iv, neg_iv, tgt_r, ctx_r, pos_v, neg_v, sem_tc, *rest):
    nbufs, sems = rest[:NBUF], rest[NBUF:]
    w = lax.axis_index("s") * NC + lax.axis_index("c")

    pltpu.sync_copy(ctx_i.at[pl.ds(w * NB, NB)], ctx_iv)
    pltpu.sync_copy(neg_i.at[pl.ds(w * NB, NB), :], neg_iv)
    pltpu.sync_copy(trows.at[pl.ds(w * NL, NL), :], tgt_r)

    handles = []
    for j in range(NTC):
        handles.append(pltpu.async_copy(
            outemb.at[ctx_iv.at[pl.ds(j * TCH, TCH)]],
            ctx_r.at[pl.ds(j * TCH, TCH)], sem_tc))

    for j in range(NBUF):
        pltpu.async_copy(outemb.at[neg_iv.at[j]], nbufs[j], sems[j])

    for h in handles:
        h.wait()

    iota = lax.iota(jnp.int32, 16)

    def pos_body(g, carry):
        acc = jnp.zeros((16,), jnp.float32)
        for rr in range(16):
            b = g * 16 + rr
            line = g * 4 + rr // 4
            col = (rr % 4) * D
            t0 = tgt_r[line, pl.ds(col, 16)]
            t1 = tgt_r[line, pl.ds(col + 16, 16)]
            c0 = ctx_r[b, pl.ds(0, 16)]
            c1 = ctx_r[b, pl.ds(16, 16)]
            s = jnp.sum(t0 * c0 + t1 * c1)
            acc = jnp.where(iota == rr, s, acc)
        pos_v[pl.ds(g * 16, 16)] = acc
        return carry

    lax.fori_loop(0, NB // 16, pos_body, 0)

    def neg_body(o, carry):
        for j in range(NBUF):
            b = o * NBUF + j
            line = o * (NBUF // 4) + j // 4
            col = (j % 4) * D
            pltpu.make_async_copy(
                outemb.at[neg_iv.at[b]], nbufs[j], sems[j]).wait()
            t0 = tgt_r[line, pl.ds(col, 16)]
            t1 = tgt_r[line, pl.ds(col + 16, 16)]
            acc0 = jnp.zeros((16,), jnp.float32)
            acc1 = jnp.zeros((16,), jnp.float32)
            for rr in range(K):
                n0 = nbufs[j][rr, pl.ds(0, 16)]
                n1 = nbufs[j][rr, pl.ds(16, 16)]
                s = jnp.sum(n0 * t0 + n1 * t1)
                if rr < 16:
                    acc0 = jnp.where(iota == rr, s, acc0)
                else:
                    acc1 = jnp.where(iota == rr - 16, s, acc1)
            neg_v[b, pl.ds(0, 16)] = acc0
            neg_v[b, pl.ds(16, 16)] = acc1
            nxt = (b + NBUF) % NB
            pltpu.async_copy(outemb.at[neg_iv.at[nxt]], nbufs[j], sems[j])
        return carry

    lax.fori_loop(0, NB // NBUF, neg_body, 0)

    for j in range(NBUF):
        pltpu.make_async_copy(
            outemb.at[neg_iv.at[j]], nbufs[j], sems[j]).wait()

    pltpu.sync_copy(pos_v, pos_o.at[pl.ds(w * NB, NB)])
    pltpu.sync_copy(neg_v, neg_o.at[pl.ds(w * NB, NB), :])


@jax.jit
def _sc_dots(ctx_idx, neg_idx, trows, out_emb):
    mesh = plsc.VectorSubcoreMesh(core_axis_name="c", subcore_axis_name="s")
    f = pl.kernel(
        _sc_body,
        mesh=mesh,
        compiler_params=pltpu.CompilerParams(
            needs_layout_passes=False, use_tc_tiling_on_sc=False),
        out_type=[
            jax.ShapeDtypeStruct((B,), jnp.float32),
            jax.ShapeDtypeStruct((B, D), jnp.float32),
        ],
        scratch_types=(
            [pltpu.VMEM((NB,), jnp.int32),
             pltpu.VMEM((NB, K), jnp.int32),
             pltpu.VMEM((NL, 128), jnp.float32),
             pltpu.VMEM((NB, D), jnp.float32),
             pltpu.VMEM((NB,), jnp.float32),
             pltpu.VMEM((NB, D), jnp.float32),
             pltpu.SemaphoreType.DMA]
            + [pltpu.VMEM((K, D), jnp.float32)] * NBUF
            + [pltpu.SemaphoreType.DMA] * NBUF
        ),
    )
    return f(ctx_idx, neg_idx, trows, out_emb)


def _tc_loss_body(pos_ref, neg_ref, o_ref):
    p = pos_ref[...]
    n = neg_ref[...]

    def ls(v):
        return jnp.minimum(v, 0.0) - jnp.log(1.0 + jnp.exp(-jnp.abs(v)))

    cols = lax.broadcasted_iota(jnp.int32, n.shape, 1)
    valid = (cols % D) < K
    s_pos = jnp.sum(ls(p))
    s_neg = jnp.sum(jnp.where(valid, ls(-n), 0.0))
    o_ref[0, 0] = -(s_pos / B + s_neg / (B * K))


@jax.jit
def _tc_loss(pos2, neg2):
    out = pl.pallas_call(
        _tc_loss_body,
        out_shape=jax.ShapeDtypeStruct((1, 1), jnp.float32),
        out_specs=pl.BlockSpec(memory_space=pltpu.SMEM),
    )(pos2, neg2)
    return out[0, 0]


def kernel(target, context, neg_samples, in_emb, out_emb):
    in_emb3 = in_emb.reshape(in_emb.shape[0] // 8, 8, D)
    trows = _sc_tgt_rows(target.astype(jnp.int32), in_emb3)
    pos1, negp = _sc_dots(context.astype(jnp.int32),
                          neg_samples.astype(jnp.int32), trows, out_emb)
    return _tc_loss(pos1.reshape(B // 128, 128),
                    negp.reshape(B * D // 128, 128))
